# baseline (device time: 48467 ns/iter reference)
import jax
import jax.numpy as jnp
from jax import lax
from jax.experimental import pallas as pl
from jax.experimental.pallas import tpu as pltpu

N_X = 2
N_Y = 2
V_LOCAL = 8192
T = 1024
D = 1024
T_HALF = T // 2
U = 8

CHUNKS = [(0, 32), (32, 32), (64, 64), (128, 96), (224, 96),
          (320, 96), (416, 64), (480, 32)]
C = len(CHUNKS)
assert sum(s for _, s in CHUNKS) == T_HALF


def kernel(ids, E):
    ids2d = ids.reshape(T, 1)

    def body(ids_smem, ids_vmem, e_hbm, out_ref,
             gbuf, xrecv, gsems, xs_sems, xr_sems, ys_sems, yr_sems):
        my_x = lax.axis_index("x")
        my_y = lax.axis_index("y")

        barrier = pltpu.get_barrier_semaphore()
        pl.semaphore_signal(barrier, inc=1, device_id=(1 - my_x, my_y),
                            device_id_type=pl.DeviceIdType.MESH)
        pl.semaphore_signal(barrier, inc=1, device_id=(my_x, 1 - my_y),
                            device_id_type=pl.DeviceIdType.MESH)
        pl.semaphore_wait(barrier, 2)

        base = my_y * T_HALF
        row0 = my_x * V_LOCAL

        def gather_issue(c):
            lo, sz = CHUNKS[c]

            def issue(j, _):
                for u in range(U):
                    i = lo + j * U + u
                    idx = ids_smem[base + i]
                    lidx = jnp.clip(idx - row0, 0, V_LOCAL - 1)
                    pltpu.make_async_copy(
                        e_hbm.at[pl.ds(lidx, 1), :],
                        gbuf.at[pl.ds(i, 1), :],
                        gsems.at[c],
                    ).start()
                return 0

            lax.fori_loop(0, sz // U, issue, 0, unroll=True)

        def gather_drain(c):
            lo, sz = CHUNKS[c]
            pltpu.make_async_copy(
                e_hbm.at[pl.ds(0, sz), :],
                gbuf.at[pl.ds(lo, sz), :],
                gsems.at[c],
            ).wait()

        def x_rdma(c):
            lo, sz = CHUNKS[c]
            return pltpu.make_async_remote_copy(
                src_ref=gbuf.at[pl.ds(lo, sz), :],
                dst_ref=xrecv.at[pl.ds(lo, sz), :],
                send_sem=xs_sems.at[c],
                recv_sem=xr_sems.at[c],
                device_id=(1 - my_x, my_y),
                device_id_type=pl.DeviceIdType.MESH,
            )

        def y_rdma(c):
            lo, sz = CHUNKS[c]
            rows = pl.ds(base + lo, sz)
            return pltpu.make_async_remote_copy(
                src_ref=out_ref.at[rows, :],
                dst_ref=out_ref.at[rows, :],
                send_sem=ys_sems.at[c],
                recv_sem=yr_sems.at[c],
                device_id=(my_x, 1 - my_y),
                device_id_type=pl.DeviceIdType.MESH,
            )

        def select_store(c):
            lo, sz = CHUNKS[c]
            idv = ids_vmem[pl.ds(base + lo, sz), :]
            own = (idv >= row0) & (idv < row0 + V_LOCAL)
            out_ref[pl.ds(base + lo, sz), :] = jnp.where(
                own, gbuf[pl.ds(lo, sz), :], xrecv[pl.ds(lo, sz), :]
            )

        for c in range(C):
            gather_issue(c)
            gather_drain(c)
            x_rdma(c).start()

        for c in range(C):
            x_rdma(c).wait_recv()
            select_store(c)
            y_rdma(c).start()

        for c in range(C):
            x_rdma(c).wait_send()
            y_rdma(c).wait_send()
            y_rdma(c).wait_recv()

    return pl.pallas_call(
        body,
        out_shape=jax.ShapeDtypeStruct((T, D), jnp.float32),
        in_specs=[
            pl.BlockSpec(memory_space=pltpu.SMEM),
            pl.BlockSpec(memory_space=pltpu.VMEM),
            pl.BlockSpec(memory_space=pl.ANY),
        ],
        out_specs=pl.BlockSpec(memory_space=pltpu.VMEM),
        scratch_shapes=[
            pltpu.VMEM((T_HALF, D), jnp.float32),
            pltpu.VMEM((T_HALF, D), jnp.float32),
            pltpu.SemaphoreType.DMA((C,)),
            pltpu.SemaphoreType.DMA((C,)),
            pltpu.SemaphoreType.DMA((C,)),
            pltpu.SemaphoreType.DMA((C,)),
            pltpu.SemaphoreType.DMA((C,)),
        ],
        compiler_params=pltpu.CompilerParams(collective_id=0),
    )(ids, ids2d, E)


# device time: 34715 ns/iter; 1.3961x vs baseline; 1.3961x over previous
import jax
import jax.numpy as jnp
from jax import lax
from jax.experimental import pallas as pl
from jax.experimental.pallas import tpu as pltpu

N_X = 2
N_Y = 2
V_LOCAL = 8192
T = 1024
D = 1024
T_HALF = T // 2
U = 8
C = 8
CH = T_HALF // C
ROW_BYTES = D * 4


def kernel(ids, E):
    def body(ids_smem, e_hbm, out_ref,
             gsems, xs_sems, xr_sems, ys_sems, yr_sems):
        my_x = lax.axis_index("x")
        my_y = lax.axis_index("y")

        barrier = pltpu.get_barrier_semaphore()
        pl.semaphore_signal(barrier, inc=1, device_id=(1 - my_x, my_y),
                            device_id_type=pl.DeviceIdType.MESH)
        pl.semaphore_signal(barrier, inc=1, device_id=(my_x, 1 - my_y),
                            device_id_type=pl.DeviceIdType.MESH)
        pl.semaphore_wait(barrier, 2)

        base = my_y * T_HALF
        row0 = my_x * V_LOCAL

        def issue_chunk(c):
            def issue(j, cnt):
                for u in range(U):
                    i = c * CH + j * U + u
                    idx = ids_smem[base + i]
                    own = (idx >= row0) & (idx < row0 + V_LOCAL)
                    lidx = jnp.clip(idx - row0, 0, V_LOCAL - 1)

                    @pl.when(own)
                    def _():
                        pltpu.make_async_copy(
                            e_hbm.at[pl.ds(lidx, 1), :],
                            out_ref.at[pl.ds(base + i, 1), :],
                            gsems.at[c],
                        ).start()
                        pltpu.make_async_remote_copy(
                            src_ref=e_hbm.at[pl.ds(lidx, 1), :],
                            dst_ref=out_ref.at[pl.ds(base + i, 1), :],
                            send_sem=xs_sems.at[c],
                            recv_sem=xr_sems.at[c],
                            device_id=(1 - my_x, my_y),
                            device_id_type=pl.DeviceIdType.MESH,
                        ).start()

                    cnt = cnt + own.astype(jnp.int32)
                return cnt

            return lax.fori_loop(0, CH // U, issue, jnp.int32(0),
                                 unroll=True)

        def y_rdma(c):
            rows = pl.ds(base + c * CH, CH)
            return pltpu.make_async_remote_copy(
                src_ref=out_ref.at[rows, :],
                dst_ref=out_ref.at[rows, :],
                send_sem=ys_sems.at[c],
                recv_sem=yr_sems.at[c],
                device_id=(my_x, 1 - my_y),
                device_id_type=pl.DeviceIdType.MESH,
            )

        def per_row(c, fn_own, fn_nbr=None):
            def step(j, _):
                for u in range(U):
                    i = c * CH + j * U + u
                    idx = ids_smem[base + i]
                    own = (idx >= row0) & (idx < row0 + V_LOCAL)
                    pl.when(own)(lambda: fn_own(i))
                    if fn_nbr is not None:
                        pl.when(jnp.logical_not(own))(lambda: fn_nbr(i))
                return 0

            lax.fori_loop(0, CH // U, step, 0, unroll=True)

        def local_wait(c, i):
            pltpu.make_async_copy(
                e_hbm.at[pl.ds(0, 1), :],
                out_ref.at[pl.ds(base + i, 1), :],
                gsems.at[c],
            ).wait()

        def xrecv_wait(c, i):
            pltpu.make_async_remote_copy(
                src_ref=e_hbm.at[pl.ds(0, 1), :],
                dst_ref=out_ref.at[pl.ds(base + i, 1), :],
                send_sem=xs_sems.at[c],
                recv_sem=xr_sems.at[c],
                device_id=(1 - my_x, my_y),
                device_id_type=pl.DeviceIdType.MESH,
            ).wait_recv()

        def xsend_wait(c, i):
            pltpu.make_async_remote_copy(
                src_ref=e_hbm.at[pl.ds(0, 1), :],
                dst_ref=out_ref.at[pl.ds(base + i, 1), :],
                send_sem=xs_sems.at[c],
                recv_sem=xr_sems.at[c],
                device_id=(1 - my_x, my_y),
                device_id_type=pl.DeviceIdType.MESH,
            ).wait_send()

        def finalize(c):
            per_row(c, lambda i: local_wait(c, i),
                    lambda i: xrecv_wait(c, i))
            y_rdma(c).start()

        for c in range(C):
            issue_chunk(c)
            if c >= 1:
                finalize(c - 1)
        finalize(C - 1)

        for c in range(C):
            per_row(c, lambda i: xsend_wait(c, i))
            y_rdma(c).wait_send()
            y_rdma(c).wait_recv()

    return pl.pallas_call(
        body,
        out_shape=jax.ShapeDtypeStruct((T, D), jnp.float32),
        in_specs=[
            pl.BlockSpec(memory_space=pltpu.SMEM),
            pl.BlockSpec(memory_space=pl.ANY),
        ],
        out_specs=pl.BlockSpec(memory_space=pltpu.VMEM),
        scratch_shapes=[
            pltpu.SemaphoreType.DMA((C,)),
            pltpu.SemaphoreType.DMA((C,)),
            pltpu.SemaphoreType.DMA((C,)),
            pltpu.SemaphoreType.DMA((C,)),
            pltpu.SemaphoreType.DMA((C,)),
        ],
        compiler_params=pltpu.CompilerParams(collective_id=0),
    )(ids, E)
